# Initial kernel scaffold; baseline (speedup 1.0000x reference)
import jax
import jax.numpy as jnp
from jax.experimental import pallas as pl


def kernel(x, y):
    raise NotImplementedError("write your pallas kernel here")


# speedup vs baseline: 1.4963x; 1.4963x over previous
import jax
import jax.numpy as jnp
from jax.experimental import pallas as pl

_B = 1024
_C = 100000
_BB = 256
_CB = 4096


def _onehot_body(y_ref, out_ref):
    j = pl.program_id(1)
    col = jax.lax.broadcasted_iota(jnp.int32, out_ref.shape, 1) + j * _CB
    out_ref[...] = (col == y_ref[...]).astype(jnp.float32)


def _onehot(y):
    y2 = y.astype(jnp.int32).reshape(_B, 1)
    return pl.pallas_call(
        _onehot_body,
        grid=(_B // _BB, pl.cdiv(_C, _CB)),
        in_specs=[pl.BlockSpec((_BB, 1), lambda i, j: (i, 0))],
        out_specs=pl.BlockSpec((_BB, _CB), lambda i, j: (i, j)),
        out_shape=jax.ShapeDtypeStruct((_B, _C), jnp.float32),
    )(y2)


def kernel(x, y):
    return (x, _onehot(y))
